# initial kernel scaffold (unmeasured)
import jax
import jax.numpy as jnp
from jax import lax
from jax.experimental import pallas as pl
from jax.experimental.pallas import tpu as pltpu

N_DEV = 8
M = 4096
N = 8192
M_CH = M // N_DEV


def kernel(x, w_mat, scale_x, scale_w):
    k_sh = x.shape[1]

    def body(x_ref, w_ref, sx_ref, sw_ref, out_ref,
             comm_ref, w_bf, send_sems, recv_sems, copy_sem):
        me = lax.axis_index("i")
        right = (me + 1) % N_DEV
        left = (me - 1) % N_DEV

        barrier_sem = pltpu.get_barrier_semaphore()
        for nbr in (left, right):
            pl.semaphore_signal(
                barrier_sem, inc=1,
                device_id=(nbr,), device_id_type=pl.DeviceIdType.MESH,
            )
        pl.semaphore_wait(barrier_sem, 2)

        w_bf[...] = w_ref[...].astype(jnp.bfloat16)

        def partial_chunk(c):
            xc = x_ref[pl.ds(c * M_CH, M_CH), :].astype(jnp.bfloat16)
            return jnp.dot(xc, w_bf[...], preferred_element_type=jnp.float32)

        comm_ref[0] = partial_chunk(me)
        for s in range(N_DEV - 1):
            ss, rs = s % 2, (s + 1) % 2
            rdma = pltpu.make_async_remote_copy(
                src_ref=comm_ref.at[ss],
                dst_ref=comm_ref.at[rs],
                send_sem=send_sems.at[ss],
                recv_sem=recv_sems.at[rs],
                device_id=(right,),
                device_id_type=pl.DeviceIdType.MESH,
            )
            rdma.start()
            rdma.wait()
            c = (me - s - 1) % N_DEV
            comm_ref[rs] = comm_ref[rs] + partial_chunk(c)

        sc = sx_ref[0] * sw_ref[0]
        y = comm_ref[1] * sc
        comm_ref[1] = y * jax.nn.sigmoid(y)

        c_own = (me + 1) % N_DEV
        cp = pltpu.make_async_copy(
            comm_ref.at[1], out_ref.at[pl.ds(c_own * M_CH, M_CH), :], copy_sem
        )
        cp.start()
        cp.wait()

        for t in range(N_DEV - 1):
            ss, rs = (t + 1) % 2, t % 2
            rdma = pltpu.make_async_remote_copy(
                src_ref=comm_ref.at[ss],
                dst_ref=comm_ref.at[rs],
                send_sem=send_sems.at[ss],
                recv_sem=recv_sems.at[rs],
                device_id=(right,),
                device_id_type=pl.DeviceIdType.MESH,
            )
            rdma.start()
            rdma.wait()
            c = (me - t) % N_DEV
            cp = pltpu.make_async_copy(
                comm_ref.at[rs], out_ref.at[pl.ds(c * M_CH, M_CH), :], copy_sem
            )
            cp.start()
            cp.wait()

    out_shape = jax.ShapeDtypeStruct((M, N), jnp.float32)
    return pl.pallas_call(
        body,
        out_shape=out_shape,
        in_specs=[
            pl.BlockSpec(memory_space=pltpu.VMEM),
            pl.BlockSpec(memory_space=pltpu.VMEM),
            pl.BlockSpec(memory_space=pltpu.SMEM),
            pl.BlockSpec(memory_space=pltpu.SMEM),
        ],
        out_specs=pl.BlockSpec(memory_space=pltpu.ANY),
        scratch_shapes=[
            pltpu.VMEM((2, M_CH, N), jnp.float32),
            pltpu.VMEM((k_sh, N), jnp.bfloat16),
            pltpu.SemaphoreType.DMA((2,)),
            pltpu.SemaphoreType.DMA((2,)),
            pltpu.SemaphoreType.DMA,
        ],
        compiler_params=pltpu.CompilerParams(collective_id=0),
    )(x, w_mat, scale_x, scale_w)


# baseline (device time: 2732290 ns/iter reference)
import jax
import jax.numpy as jnp
from jax import lax
from jax.experimental import pallas as pl
from jax.experimental.pallas import tpu as pltpu

N_DEV = 8
M = 4096
N = 8192
M_CH = M // N_DEV
N_STRIP = 2048


def kernel(x, w_mat, scale_x, scale_w):
    k_sh = x.shape[1]

    def body(x_ref, w_ref, sx_ref, sw_ref, out_ref,
             comm_ref, w_bf, w_stage, send_sems, recv_sems, copy_sem):
        me = lax.axis_index("i")
        right = (me + 1) % N_DEV
        left = (me - 1) % N_DEV

        barrier_sem = pltpu.get_barrier_semaphore()
        for nbr in (left, right):
            pl.semaphore_signal(
                barrier_sem, inc=1,
                device_id=(nbr,), device_id_type=pl.DeviceIdType.MESH,
            )
        pl.semaphore_wait(barrier_sem, 2)

        for j in range(N // N_STRIP):
            cols = pl.ds(j * N_STRIP, N_STRIP)
            cp = pltpu.make_async_copy(w_ref.at[:, cols], w_stage, copy_sem)
            cp.start()
            cp.wait()
            w_bf[:, cols] = w_stage[...].astype(jnp.bfloat16)

        def accum_chunk(c, slot, first):
            xc = x_ref[pl.ds(c * M_CH, M_CH), :].astype(jnp.bfloat16)
            for j in range(N // N_STRIP):
                cols = pl.ds(j * N_STRIP, N_STRIP)
                p = jnp.dot(xc, w_bf[:, cols],
                            preferred_element_type=jnp.float32)
                if first:
                    comm_ref[slot, :, cols] = p
                else:
                    comm_ref[slot, :, cols] = comm_ref[slot, :, cols] + p

        accum_chunk(me, 0, first=True)
        for s in range(N_DEV - 1):
            ss, rs = s % 2, (s + 1) % 2
            rdma = pltpu.make_async_remote_copy(
                src_ref=comm_ref.at[ss],
                dst_ref=comm_ref.at[rs],
                send_sem=send_sems.at[ss],
                recv_sem=recv_sems.at[rs],
                device_id=(right,),
                device_id_type=pl.DeviceIdType.MESH,
            )
            rdma.start()
            rdma.wait()
            accum_chunk((me - s - 1) % N_DEV, rs, first=False)

        sc = sx_ref[0] * sw_ref[0]
        for j in range(N // N_STRIP):
            cols = pl.ds(j * N_STRIP, N_STRIP)
            y = comm_ref[1, :, cols] * sc
            comm_ref[1, :, cols] = y * jax.nn.sigmoid(y)

        c_own = (me + 1) % N_DEV
        cp = pltpu.make_async_copy(
            comm_ref.at[1], out_ref.at[pl.ds(c_own * M_CH, M_CH), :], copy_sem
        )
        cp.start()
        cp.wait()

        for t in range(N_DEV - 1):
            ss, rs = (t + 1) % 2, t % 2
            rdma = pltpu.make_async_remote_copy(
                src_ref=comm_ref.at[ss],
                dst_ref=comm_ref.at[rs],
                send_sem=send_sems.at[ss],
                recv_sem=recv_sems.at[rs],
                device_id=(right,),
                device_id_type=pl.DeviceIdType.MESH,
            )
            rdma.start()
            rdma.wait()
            c = (me - t) % N_DEV
            cp = pltpu.make_async_copy(
                comm_ref.at[rs], out_ref.at[pl.ds(c * M_CH, M_CH), :], copy_sem
            )
            cp.start()
            cp.wait()

    out_shape = jax.ShapeDtypeStruct((M, N), jnp.float32)
    return pl.pallas_call(
        body,
        out_shape=out_shape,
        in_specs=[
            pl.BlockSpec(memory_space=pltpu.VMEM),
            pl.BlockSpec(memory_space=pl.ANY),
            pl.BlockSpec(memory_space=pltpu.SMEM),
            pl.BlockSpec(memory_space=pltpu.SMEM),
        ],
        out_specs=pl.BlockSpec(memory_space=pl.ANY),
        scratch_shapes=[
            pltpu.VMEM((2, M_CH, N), jnp.float32),
            pltpu.VMEM((k_sh, N), jnp.bfloat16),
            pltpu.VMEM((k_sh, N_STRIP), jnp.float32),
            pltpu.SemaphoreType.DMA((2,)),
            pltpu.SemaphoreType.DMA((2,)),
            pltpu.SemaphoreType.DMA,
        ],
        compiler_params=pltpu.CompilerParams(
            collective_id=0, vmem_limit_bytes=60 * 1024 * 1024
        ),
    )(x, w_mat, scale_x, scale_w)


# device time: 812804 ns/iter; 3.3616x vs baseline; 3.3616x over previous
import jax
import jax.numpy as jnp
from jax import lax
from jax.experimental import pallas as pl
from jax.experimental.pallas import tpu as pltpu

N_DEV = 8
M = 4096
N = 8192
M_CH = M // N_DEV
N_HALF = N // 2
N_STRIP = 2048


def kernel(x, w_mat, scale_x, scale_w):
    k_sh = x.shape[1]

    def body(x_ref, w_ref, sx_ref, sw_ref, out_ref,
             comm_ref, w_bf, w_stage, p_stage, out_stage,
             r_send, r_recv, l_send, l_recv, copy_sem):
        me = lax.axis_index("i")
        right = (me + 1) % N_DEV
        left = (me - 1) % N_DEV

        barrier_sem = pltpu.get_barrier_semaphore()
        for nbr in (left, right):
            pl.semaphore_signal(
                barrier_sem, inc=1,
                device_id=(nbr,), device_id_type=pl.DeviceIdType.MESH,
            )
        pl.semaphore_wait(barrier_sem, 2)

        for j in range(N // N_STRIP):
            cols = pl.ds(j * N_STRIP, N_STRIP)
            cp = pltpu.make_async_copy(w_ref.at[:, cols], w_stage, copy_sem)
            cp.start()
            cp.wait()
            w_bf[:, cols] = w_stage[...].astype(jnp.bfloat16)

        def partial_half(c, h, dst):
            xc = x_ref[pl.ds(c * M_CH, M_CH), :].astype(jnp.bfloat16)
            for j in range(N_HALF // N_STRIP):
                cols = pl.ds(j * N_STRIP, N_STRIP)
                wcols = pl.ds(h * N_HALF + j * N_STRIP, N_STRIP)
                p = jnp.dot(xc, w_bf[:, wcols],
                            preferred_element_type=jnp.float32)
                dst[:, cols] = p.astype(jnp.bfloat16)

        def make_hop(ss, rs):
            rr = pltpu.make_async_remote_copy(
                src_ref=comm_ref.at[ss, 0], dst_ref=comm_ref.at[rs, 0],
                send_sem=r_send.at[ss], recv_sem=r_recv.at[rs],
                device_id=(right,), device_id_type=pl.DeviceIdType.MESH,
            )
            ll = pltpu.make_async_remote_copy(
                src_ref=comm_ref.at[ss, 1], dst_ref=comm_ref.at[rs, 1],
                send_sem=l_send.at[ss], recv_sem=l_recv.at[rs],
                device_id=(left,), device_id_type=pl.DeviceIdType.MESH,
            )
            return rr, ll

        partial_half(me, 0, comm_ref.at[0, 0])
        partial_half(me, 1, comm_ref.at[0, 1])
        for s in range(N_DEV - 1):
            ss, rs = s % 2, (s + 1) % 2
            rr, ll = make_hop(ss, rs)
            rr.start()
            ll.start()
            partial_half((me - s - 1) % N_DEV, 0, p_stage.at[0])
            partial_half((me + s + 1) % N_DEV, 1, p_stage.at[1])
            rr.wait()
            ll.wait()
            for h in (0, 1):
                for j in range(N_HALF // N_STRIP):
                    cols = pl.ds(j * N_STRIP, N_STRIP)
                    acc = (comm_ref[rs, h, :, cols].astype(jnp.float32)
                           + p_stage[h, :, cols].astype(jnp.float32))
                    comm_ref[rs, h, :, cols] = acc.astype(jnp.bfloat16)

        sc = sx_ref[0] * sw_ref[0]
        own = ((me + 1) % N_DEV, (me - 1) % N_DEV)
        for h in (0, 1):
            for j in range(N_HALF // N_STRIP):
                cols = pl.ds(j * N_STRIP, N_STRIP)
                y = comm_ref[1, h, :, cols].astype(jnp.float32) * sc
                r = y * jax.nn.sigmoid(y)
                comm_ref[1, h, :, cols] = r.astype(jnp.bfloat16)
                out_stage[:, cols] = r
            cp = pltpu.make_async_copy(
                out_stage,
                out_ref.at[pl.ds(own[h] * M_CH, M_CH),
                           pl.ds(h * N_HALF, N_HALF)],
                copy_sem,
            )
            cp.start()
            cp.wait()

        rr, ll = make_hop(1, 0)
        rr.start()
        ll.start()
        for t in range(N_DEV - 1):
            ss, rs = (t + 1) % 2, t % 2
            rr.wait()
            ll.wait()
            if t < N_DEV - 2:
                rr, ll = make_hop(rs, ss)
                rr.start()
                ll.start()
            rows = ((me - t) % N_DEV, (me + t) % N_DEV)
            for h in (0, 1):
                for j in range(N_HALF // N_STRIP):
                    cols = pl.ds(j * N_STRIP, N_STRIP)
                    out_stage[:, cols] = comm_ref[
                        rs, h, :, cols].astype(jnp.float32)
                cp = pltpu.make_async_copy(
                    out_stage,
                    out_ref.at[pl.ds(rows[h] * M_CH, M_CH),
                               pl.ds(h * N_HALF, N_HALF)],
                    copy_sem,
                )
                cp.start()
                cp.wait()

    out_shape = jax.ShapeDtypeStruct((M, N), jnp.float32)
    return pl.pallas_call(
        body,
        out_shape=out_shape,
        in_specs=[
            pl.BlockSpec(memory_space=pltpu.VMEM),
            pl.BlockSpec(memory_space=pl.ANY),
            pl.BlockSpec(memory_space=pltpu.SMEM),
            pl.BlockSpec(memory_space=pltpu.SMEM),
        ],
        out_specs=pl.BlockSpec(memory_space=pl.ANY),
        scratch_shapes=[
            pltpu.VMEM((2, 2, M_CH, N_HALF), jnp.bfloat16),
            pltpu.VMEM((k_sh, N), jnp.bfloat16),
            pltpu.VMEM((k_sh, N_STRIP), jnp.float32),
            pltpu.VMEM((2, M_CH, N_HALF), jnp.bfloat16),
            pltpu.VMEM((M_CH, N_HALF), jnp.float32),
            pltpu.SemaphoreType.DMA((2,)),
            pltpu.SemaphoreType.DMA((2,)),
            pltpu.SemaphoreType.DMA((2,)),
            pltpu.SemaphoreType.DMA((2,)),
            pltpu.SemaphoreType.DMA,
        ],
        compiler_params=pltpu.CompilerParams(
            collective_id=0, vmem_limit_bytes=60 * 1024 * 1024
        ),
    )(x, w_mat, scale_x, scale_w)


# device time: 783333 ns/iter; 3.4880x vs baseline; 1.0376x over previous
import jax
import jax.numpy as jnp
from jax import lax
from jax.experimental import pallas as pl
from jax.experimental.pallas import tpu as pltpu

N_DEV = 8
M = 4096
N = 8192
M_CH = M // N_DEV
N_HALF = N // 2
N_SUB = 2
N_SUB_W = N_HALF // N_SUB
N_STRIP = 2048


def kernel(x, w_mat, scale_x, scale_w):
    k_sh = x.shape[1]

    def body(x_ref, w_ref, sx_ref, sw_ref, out_ref,
             comm_ref, w_bf, w_stage, p_stage, out_stage,
             r_send, r_recv, l_send, l_recv, copy_sems):
        me = lax.axis_index("i")
        right = (me + 1) % N_DEV
        left = (me - 1) % N_DEV

        barrier_sem = pltpu.get_barrier_semaphore()
        for nbr in (left, right):
            pl.semaphore_signal(
                barrier_sem, inc=1,
                device_id=(nbr,), device_id_type=pl.DeviceIdType.MESH,
            )
        pl.semaphore_wait(barrier_sem, 2)

        for j in range(N // N_STRIP):
            cols = pl.ds(j * N_STRIP, N_STRIP)
            cp = pltpu.make_async_copy(w_ref.at[:, cols], w_stage,
                                       copy_sems.at[0])
            cp.start()
            cp.wait()
            w_bf[:, cols] = w_stage[...].astype(jnp.bfloat16)

        def partial_half(c, h, dst):
            xc = x_ref[pl.ds(c * M_CH, M_CH), :].astype(jnp.bfloat16)
            for j in range(N_HALF // N_STRIP):
                cols = pl.ds(j * N_STRIP, N_STRIP)
                wcols = pl.ds(h * N_HALF + j * N_STRIP, N_STRIP)
                p = jnp.dot(xc, w_bf[:, wcols],
                            preferred_element_type=jnp.float32)
                dst[:, cols] = p.astype(jnp.bfloat16)

        def make_sub(ss, rs, k):
            cols = pl.ds(k * N_SUB_W, N_SUB_W)
            rr = pltpu.make_async_remote_copy(
                src_ref=comm_ref.at[ss, 0, :, cols],
                dst_ref=comm_ref.at[rs, 0, :, cols],
                send_sem=r_send.at[ss, k], recv_sem=r_recv.at[rs, k],
                device_id=(right,), device_id_type=pl.DeviceIdType.MESH,
            )
            ll = pltpu.make_async_remote_copy(
                src_ref=comm_ref.at[ss, 1, :, cols],
                dst_ref=comm_ref.at[rs, 1, :, cols],
                send_sem=l_send.at[ss, k], recv_sem=l_recv.at[rs, k],
                device_id=(left,), device_id_type=pl.DeviceIdType.MESH,
            )
            return rr, ll

        def start_hop(ss, rs):
            subs = []
            for k in range(N_SUB):
                rr, ll = make_sub(ss, rs, k)
                rr.start()
                ll.start()
                subs.append((rr, ll))
            return subs

        partial_half(me, 0, comm_ref.at[0, 0])
        partial_half(me, 1, comm_ref.at[0, 1])
        for s in range(N_DEV - 1):
            ss, rs = s % 2, (s + 1) % 2
            subs = start_hop(ss, rs)
            partial_half((me - s - 1) % N_DEV, 0, p_stage.at[0])
            partial_half((me + s + 1) % N_DEV, 1, p_stage.at[1])
            for k in range(N_SUB):
                rr, ll = subs[k]
                rr.wait()
                ll.wait()
                cols = pl.ds(k * N_SUB_W, N_SUB_W)
                for h in (0, 1):
                    comm_ref[rs, h, :, cols] = (
                        comm_ref[rs, h, :, cols] + p_stage[h, :, cols])

        sc = sx_ref[0] * sw_ref[0]
        for h in (0, 1):
            for j in range(N_HALF // N_STRIP):
                cols = pl.ds(j * N_STRIP, N_STRIP)
                y = comm_ref[1, h, :, cols].astype(jnp.float32) * sc
                r = y * jax.nn.sigmoid(y)
                comm_ref[1, h, :, cols] = r.astype(jnp.bfloat16)

        subs = start_hop(1, 0)

        def write_out(slot, rows_by_half):
            i = 0
            cps = []
            for h in (0, 1):
                rows = pl.ds(rows_by_half[h] * M_CH, M_CH)
                for j in range(N_HALF // N_STRIP):
                    cols = pl.ds(j * N_STRIP, N_STRIP)
                    if len(cps) >= 2:
                        cps[-2].wait()
                    out_stage[i % 2] = comm_ref[
                        slot, h, :, cols].astype(jnp.float32)
                    cp = pltpu.make_async_copy(
                        out_stage.at[i % 2],
                        out_ref.at[rows,
                                   pl.ds(h * N_HALF + j * N_STRIP, N_STRIP)],
                        copy_sems.at[i % 2],
                    )
                    cp.start()
                    cps.append(cp)
                    i += 1
            cps[-2].wait()
            cps[-1].wait()

        write_out(1, ((me + 1) % N_DEV, (me - 1) % N_DEV))

        for t in range(N_DEV - 1):
            ss, rs = (t + 1) % 2, t % 2
            for rr, ll in subs:
                rr.wait()
                ll.wait()
            if t < N_DEV - 2:
                subs = start_hop(rs, ss)
            write_out(rs, ((me - t) % N_DEV, (me + t) % N_DEV))

    out_shape = jax.ShapeDtypeStruct((M, N), jnp.float32)
    return pl.pallas_call(
        body,
        out_shape=out_shape,
        in_specs=[
            pl.BlockSpec(memory_space=pltpu.VMEM),
            pl.BlockSpec(memory_space=pl.ANY),
            pl.BlockSpec(memory_space=pltpu.SMEM),
            pl.BlockSpec(memory_space=pltpu.SMEM),
        ],
        out_specs=pl.BlockSpec(memory_space=pl.ANY),
        scratch_shapes=[
            pltpu.VMEM((2, 2, M_CH, N_HALF), jnp.bfloat16),
            pltpu.VMEM((k_sh, N), jnp.bfloat16),
            pltpu.VMEM((k_sh, N_STRIP), jnp.float32),
            pltpu.VMEM((2, M_CH, N_HALF), jnp.bfloat16),
            pltpu.VMEM((2, M_CH, N_STRIP), jnp.float32),
            pltpu.SemaphoreType.DMA((2, N_SUB)),
            pltpu.SemaphoreType.DMA((2, N_SUB)),
            pltpu.SemaphoreType.DMA((2, N_SUB)),
            pltpu.SemaphoreType.DMA((2, N_SUB)),
            pltpu.SemaphoreType.DMA((2,)),
        ],
        compiler_params=pltpu.CompilerParams(
            collective_id=0, vmem_limit_bytes=60 * 1024 * 1024
        ),
    )(x, w_mat, scale_x, scale_w)


# device time: 778789 ns/iter; 3.5084x vs baseline; 1.0058x over previous
import jax
import jax.numpy as jnp
from jax import lax
from jax.experimental import pallas as pl
from jax.experimental.pallas import tpu as pltpu

N_DEV = 8
M = 4096
N = 8192
M_CH = M // N_DEV
N_HALF = N // 2
N_SUB = 4
SUBW = N_HALF // N_SUB


def kernel(x, w_mat, scale_x, scale_w):
    k_sh = x.shape[1]

    def body(x_ref, w_ref, sx_ref, sw_ref, out_ref,
             comm_ref, w_bf, w_stage, p_stage, out_stage,
             r_send, r_recv, l_send, l_recv, copy_sems):
        me = lax.axis_index("i")
        right = (me + 1) % N_DEV
        left = (me - 1) % N_DEV

        barrier_sem = pltpu.get_barrier_semaphore()
        for nbr in (left, right):
            pl.semaphore_signal(
                barrier_sem, inc=1,
                device_id=(nbr,), device_id_type=pl.DeviceIdType.MESH,
            )
        pl.semaphore_wait(barrier_sem, 2)

        def make_dir(ss, rs, h, k):
            cols = pl.ds(k * SUBW, SUBW)
            send_sems, recv_sems = (r_send, r_recv) if h == 0 else (
                l_send, l_recv)
            tgt = right if h == 0 else left
            return pltpu.make_async_remote_copy(
                src_ref=comm_ref.at[ss, h, :, cols],
                dst_ref=comm_ref.at[rs, h, :, cols],
                send_sem=send_sems.at[ss, k], recv_sem=recv_sems.at[rs, k],
                device_id=(tgt,), device_id_type=pl.DeviceIdType.MESH,
            )

        def start_hop(ss, rs):
            subs = {}
            for h in (0, 1):
                for k in range(N_SUB):
                    d = make_dir(ss, rs, h, k)
                    d.start()
                    subs[(h, k)] = d
            return subs

        xme = x_ref[pl.ds(me * M_CH, M_CH), :].astype(jnp.bfloat16)
        subs = {}
        for h in (0, 1):
            for k in range(N_SUB):
                wcols = pl.ds(h * N_HALF + k * SUBW, SUBW)
                cp = pltpu.make_async_copy(
                    w_ref.at[:, wcols], w_stage, copy_sems.at[0])
                cp.start()
                cp.wait()
                w_bf[:, wcols] = w_stage[...].astype(jnp.bfloat16)
                comm_ref[0, h, :, pl.ds(k * SUBW, SUBW)] = jnp.dot(
                    xme, w_bf[:, wcols],
                    preferred_element_type=jnp.float32).astype(jnp.bfloat16)
                d = make_dir(0, 1, h, k)
                d.start()
                subs[(h, k)] = d

        def partial_half(c, h, dst):
            xc = x_ref[pl.ds(c * M_CH, M_CH), :].astype(jnp.bfloat16)
            for j in range(N_SUB):
                cols = pl.ds(j * SUBW, SUBW)
                wcols = pl.ds(h * N_HALF + j * SUBW, SUBW)
                p = jnp.dot(xc, w_bf[:, wcols],
                            preferred_element_type=jnp.float32)
                dst[:, cols] = p.astype(jnp.bfloat16)

        for s in range(N_DEV - 1):
            ss, rs = s % 2, (s + 1) % 2
            if s > 0:
                subs = start_hop(ss, rs)
            partial_half((me - s - 1) % N_DEV, 0, p_stage.at[0])
            partial_half((me + s + 1) % N_DEV, 1, p_stage.at[1])
            for k in range(N_SUB):
                subs[(0, k)].wait()
                subs[(1, k)].wait()
                cols = pl.ds(k * SUBW, SUBW)
                for h in (0, 1):
                    comm_ref[rs, h, :, cols] = (
                        comm_ref[rs, h, :, cols] + p_stage[h, :, cols])

        sc = sx_ref[0] * sw_ref[0]
        for h in (0, 1):
            for j in range(N_SUB):
                cols = pl.ds(j * SUBW, SUBW)
                y = comm_ref[1, h, :, cols].astype(jnp.float32) * sc
                r = y * jax.nn.sigmoid(y)
                comm_ref[1, h, :, cols] = r.astype(jnp.bfloat16)

        subs = start_hop(1, 0)

        def write_out(slot, rows_by_half):
            i = 0
            cps = []
            for h in (0, 1):
                rows = pl.ds(rows_by_half[h] * M_CH, M_CH)
                for j in range(N_SUB):
                    cols = pl.ds(j * SUBW, SUBW)
                    if len(cps) >= 2:
                        cps[-2].wait()
                    out_stage[i % 2] = comm_ref[
                        slot, h, :, cols].astype(jnp.float32)
                    cp = pltpu.make_async_copy(
                        out_stage.at[i % 2],
                        out_ref.at[rows, pl.ds(h * N_HALF + j * SUBW, SUBW)],
                        copy_sems.at[i % 2],
                    )
                    cp.start()
                    cps.append(cp)
                    i += 1
            cps[-2].wait()
            cps[-1].wait()

        write_out(1, ((me + 1) % N_DEV, (me - 1) % N_DEV))

        for t in range(N_DEV - 1):
            ss, rs = (t + 1) % 2, t % 2
            for k in range(N_SUB):
                subs[(0, k)].wait()
                subs[(1, k)].wait()
            if t < N_DEV - 2:
                subs = start_hop(rs, ss)
            write_out(rs, ((me - t) % N_DEV, (me + t) % N_DEV))

    out_shape = jax.ShapeDtypeStruct((M, N), jnp.float32)
    return pl.pallas_call(
        body,
        out_shape=out_shape,
        in_specs=[
            pl.BlockSpec(memory_space=pltpu.VMEM),
            pl.BlockSpec(memory_space=pl.ANY),
            pl.BlockSpec(memory_space=pltpu.SMEM),
            pl.BlockSpec(memory_space=pltpu.SMEM),
        ],
        out_specs=pl.BlockSpec(memory_space=pl.ANY),
        scratch_shapes=[
            pltpu.VMEM((2, 2, M_CH, N_HALF), jnp.bfloat16),
            pltpu.VMEM((k_sh, N), jnp.bfloat16),
            pltpu.VMEM((k_sh, SUBW), jnp.float32),
            pltpu.VMEM((2, M_CH, N_HALF), jnp.bfloat16),
            pltpu.VMEM((2, M_CH, SUBW), jnp.float32),
            pltpu.SemaphoreType.DMA((2, N_SUB)),
            pltpu.SemaphoreType.DMA((2, N_SUB)),
            pltpu.SemaphoreType.DMA((2, N_SUB)),
            pltpu.SemaphoreType.DMA((2, N_SUB)),
            pltpu.SemaphoreType.DMA((2,)),
        ],
        compiler_params=pltpu.CompilerParams(
            collective_id=0, vmem_limit_bytes=60 * 1024 * 1024
        ),
    )(x, w_mat, scale_x, scale_w)


# device time: 731705 ns/iter; 3.7341x vs baseline; 1.0643x over previous
import jax
import jax.numpy as jnp
from jax import lax
from jax.experimental import pallas as pl
from jax.experimental.pallas import tpu as pltpu

N_DEV = 8
M = 4096
N = 8192
M_CH = M // N_DEV
N_HALF = N // 2
N_SUB = 4
SUBW = N_HALF // N_SUB


def kernel(x, w_mat, scale_x, scale_w):
    k_sh = x.shape[1]

    def body(x_ref, w_ref, sx_ref, sw_ref, out_ref,
             comm_ref, w_bf, w_stage, p_stage, out_stage,
             r_send, r_recv, l_send, l_recv, copy_sems):
        me = lax.axis_index("i")
        right = (me + 1) % N_DEV
        left = (me - 1) % N_DEV

        barrier_sem = pltpu.get_barrier_semaphore()
        for nbr in (left, right):
            pl.semaphore_signal(
                barrier_sem, inc=1,
                device_id=(nbr,), device_id_type=pl.DeviceIdType.MESH,
            )
        pl.semaphore_wait(barrier_sem, 2)

        def make_dir(ss, rs, h, k):
            cols = pl.ds(k * SUBW, SUBW)
            send_sems, recv_sems = (r_send, r_recv) if h == 0 else (
                l_send, l_recv)
            tgt = right if h == 0 else left
            return pltpu.make_async_remote_copy(
                src_ref=comm_ref.at[ss, h, :, cols],
                dst_ref=comm_ref.at[rs, h, :, cols],
                send_sem=send_sems.at[ss, k], recv_sem=recv_sems.at[rs, k],
                device_id=(tgt,), device_id_type=pl.DeviceIdType.MESH,
            )

        xme = x_ref[pl.ds(me * M_CH, M_CH), :].astype(jnp.bfloat16)
        subs = {}
        for k in range(N_SUB):
            for h in (0, 1):
                wcols = pl.ds(h * N_HALF + k * SUBW, SUBW)
                cp = pltpu.make_async_copy(
                    w_ref.at[:, wcols], w_stage, copy_sems.at[0])
                cp.start()
                cp.wait()
                w_bf[:, wcols] = w_stage[...].astype(jnp.bfloat16)
                comm_ref[0, h, :, pl.ds(k * SUBW, SUBW)] = jnp.dot(
                    xme, w_bf[:, wcols],
                    preferred_element_type=jnp.float32).astype(jnp.bfloat16)
                d = make_dir(0, 1, h, k)
                d.start()
                subs[(h, k)] = d

        def partial_half(c, h, dst):
            xc = x_ref[pl.ds(c * M_CH, M_CH), :].astype(jnp.bfloat16)
            for j in range(N_SUB):
                cols = pl.ds(j * SUBW, SUBW)
                wcols = pl.ds(h * N_HALF + j * SUBW, SUBW)
                p = jnp.dot(xc, w_bf[:, wcols],
                            preferred_element_type=jnp.float32)
                dst[:, cols] = p.astype(jnp.bfloat16)

        sc = sx_ref[0] * sw_ref[0]

        for s in range(N_DEV - 1):
            ss, rs = s % 2, (s + 1) % 2
            partial_half((me - s - 1) % N_DEV, 0, p_stage.at[0])
            partial_half((me + s + 1) % N_DEV, 1, p_stage.at[1])
            nxt = {}
            for k in range(N_SUB):
                subs[(0, k)].wait()
                subs[(1, k)].wait()
                cols = pl.ds(k * SUBW, SUBW)
                for h in (0, 1):
                    comm_ref[rs, h, :, cols] = (
                        comm_ref[rs, h, :, cols] + p_stage[h, :, cols])
                if s == N_DEV - 2:
                    for h in (0, 1):
                        y = comm_ref[1, h, :, cols].astype(jnp.float32) * sc
                        r = y * jax.nn.sigmoid(y)
                        comm_ref[1, h, :, cols] = r.astype(jnp.bfloat16)
                for h in (0, 1):
                    d = make_dir(rs, ss, h, k)
                    d.start()
                    nxt[(h, k)] = d
            subs = nxt

        pend = []
        pp = [0]

        def write_strip(slot, h, k, row_chunk):
            if len(pend) >= 2:
                pend[-2].wait()
            i = pp[0]
            out_stage[i % 2] = comm_ref[
                slot, h, :, pl.ds(k * SUBW, SUBW)].astype(jnp.float32)
            cp = pltpu.make_async_copy(
                out_stage.at[i % 2],
                out_ref.at[pl.ds(row_chunk * M_CH, M_CH),
                           pl.ds(h * N_HALF + k * SUBW, SUBW)],
                copy_sems.at[i % 2],
            )
            cp.start()
            pend.append(cp)
            pp[0] = i + 1

        for k in range(N_SUB):
            write_strip(1, 0, k, (me + 1) % N_DEV)
            write_strip(1, 1, k, (me - 1) % N_DEV)

        for t in range(N_DEV - 1):
            ss, rs = (t + 1) % 2, t % 2
            nxt = {}
            for k in range(N_SUB):
                subs[(0, k)].wait()
                subs[(1, k)].wait()
                if t < N_DEV - 2:
                    for h in (0, 1):
                        d = make_dir(rs, ss, h, k)
                        d.start()
                        nxt[(h, k)] = d
                write_strip(rs, 0, k, (me - t) % N_DEV)
                write_strip(rs, 1, k, (me + t) % N_DEV)
            subs = nxt

        pend[-2].wait()
        pend[-1].wait()

    out_shape = jax.ShapeDtypeStruct((M, N), jnp.float32)
    return pl.pallas_call(
        body,
        out_shape=out_shape,
        in_specs=[
            pl.BlockSpec(memory_space=pltpu.VMEM),
            pl.BlockSpec(memory_space=pl.ANY),
            pl.BlockSpec(memory_space=pltpu.SMEM),
            pl.BlockSpec(memory_space=pltpu.SMEM),
        ],
        out_specs=pl.BlockSpec(memory_space=pl.ANY),
        scratch_shapes=[
            pltpu.VMEM((2, 2, M_CH, N_HALF), jnp.bfloat16),
            pltpu.VMEM((k_sh, N), jnp.bfloat16),
            pltpu.VMEM((k_sh, SUBW), jnp.float32),
            pltpu.VMEM((2, M_CH, N_HALF), jnp.bfloat16),
            pltpu.VMEM((2, M_CH, SUBW), jnp.float32),
            pltpu.SemaphoreType.DMA((2, N_SUB)),
            pltpu.SemaphoreType.DMA((2, N_SUB)),
            pltpu.SemaphoreType.DMA((2, N_SUB)),
            pltpu.SemaphoreType.DMA((2, N_SUB)),
            pltpu.SemaphoreType.DMA((2,)),
        ],
        compiler_params=pltpu.CompilerParams(
            collective_id=0, vmem_limit_bytes=60 * 1024 * 1024
        ),
    )(x, w_mat, scale_x, scale_w)


# device time: 730186 ns/iter; 3.7419x vs baseline; 1.0021x over previous
import jax
import jax.numpy as jnp
from jax import lax
from jax.experimental import pallas as pl
from jax.experimental.pallas import tpu as pltpu

N_DEV = 8
M = 4096
N = 8192
M_CH = M // N_DEV
N_HALF = N // 2
N_SUB = 8
SUBW = N_HALF // N_SUB


def kernel(x, w_mat, scale_x, scale_w):
    k_sh = x.shape[1]

    def body(x_ref, w_ref, sx_ref, sw_ref, out_ref,
             comm_ref, w_bf, w_stage, p_stage, out_stage,
             r_send, r_recv, l_send, l_recv, copy_sems):
        me = lax.axis_index("i")
        right = (me + 1) % N_DEV
        left = (me - 1) % N_DEV

        barrier_sem = pltpu.get_barrier_semaphore()
        for nbr in (left, right):
            pl.semaphore_signal(
                barrier_sem, inc=1,
                device_id=(nbr,), device_id_type=pl.DeviceIdType.MESH,
            )
        pl.semaphore_wait(barrier_sem, 2)

        def make_dir(ss, rs, h, k):
            cols = pl.ds(k * SUBW, SUBW)
            send_sems, recv_sems = (r_send, r_recv) if h == 0 else (
                l_send, l_recv)
            tgt = right if h == 0 else left
            return pltpu.make_async_remote_copy(
                src_ref=comm_ref.at[ss, h, :, cols],
                dst_ref=comm_ref.at[rs, h, :, cols],
                send_sem=send_sems.at[ss, k], recv_sem=recv_sems.at[rs, k],
                device_id=(tgt,), device_id_type=pl.DeviceIdType.MESH,
            )

        xme = x_ref[pl.ds(me * M_CH, M_CH), :].astype(jnp.bfloat16)
        subs = {}
        for k in range(N_SUB):
            for h in (0, 1):
                wcols = pl.ds(h * N_HALF + k * SUBW, SUBW)
                cp = pltpu.make_async_copy(
                    w_ref.at[:, wcols], w_stage, copy_sems.at[0])
                cp.start()
                cp.wait()
                w_bf[:, wcols] = w_stage[...].astype(jnp.bfloat16)
                comm_ref[0, h, :, pl.ds(k * SUBW, SUBW)] = jnp.dot(
                    xme, w_bf[:, wcols],
                    preferred_element_type=jnp.float32).astype(jnp.bfloat16)
                d = make_dir(0, 1, h, k)
                d.start()
                subs[(h, k)] = d

        def partial_half(c, h, dst):
            xc = x_ref[pl.ds(c * M_CH, M_CH), :].astype(jnp.bfloat16)
            for j in range(N_SUB):
                cols = pl.ds(j * SUBW, SUBW)
                wcols = pl.ds(h * N_HALF + j * SUBW, SUBW)
                p = jnp.dot(xc, w_bf[:, wcols],
                            preferred_element_type=jnp.float32)
                dst[:, cols] = p.astype(jnp.bfloat16)

        sc = sx_ref[0] * sw_ref[0]

        for s in range(N_DEV - 1):
            ss, rs = s % 2, (s + 1) % 2
            partial_half((me - s - 1) % N_DEV, 0, p_stage.at[0])
            partial_half((me + s + 1) % N_DEV, 1, p_stage.at[1])
            nxt = {}
            for k in range(N_SUB):
                subs[(0, k)].wait()
                subs[(1, k)].wait()
                cols = pl.ds(k * SUBW, SUBW)
                for h in (0, 1):
                    comm_ref[rs, h, :, cols] = (
                        comm_ref[rs, h, :, cols] + p_stage[h, :, cols])
                if s == N_DEV - 2:
                    for h in (0, 1):
                        y = comm_ref[1, h, :, cols].astype(jnp.float32) * sc
                        r = y * jax.nn.sigmoid(y)
                        comm_ref[1, h, :, cols] = r.astype(jnp.bfloat16)
                for h in (0, 1):
                    d = make_dir(rs, ss, h, k)
                    d.start()
                    nxt[(h, k)] = d
            subs = nxt

        pend = []
        pp = [0]

        def write_strip(slot, h, k, row_chunk):
            if len(pend) >= 2:
                pend[-2].wait()
            i = pp[0]
            out_stage[i % 2] = comm_ref[
                slot, h, :, pl.ds(k * SUBW, SUBW)].astype(jnp.float32)
            cp = pltpu.make_async_copy(
                out_stage.at[i % 2],
                out_ref.at[pl.ds(row_chunk * M_CH, M_CH),
                           pl.ds(h * N_HALF + k * SUBW, SUBW)],
                copy_sems.at[i % 2],
            )
            cp.start()
            pend.append(cp)
            pp[0] = i + 1

        for k in range(N_SUB):
            write_strip(1, 0, k, (me + 1) % N_DEV)
            write_strip(1, 1, k, (me - 1) % N_DEV)

        for t in range(N_DEV - 1):
            ss, rs = (t + 1) % 2, t % 2
            nxt = {}
            for k in range(N_SUB):
                subs[(0, k)].wait()
                subs[(1, k)].wait()
                if t < N_DEV - 2:
                    for h in (0, 1):
                        d = make_dir(rs, ss, h, k)
                        d.start()
                        nxt[(h, k)] = d
                write_strip(rs, 0, k, (me - t) % N_DEV)
                write_strip(rs, 1, k, (me + t) % N_DEV)
            subs = nxt

        pend[-2].wait()
        pend[-1].wait()

    out_shape = jax.ShapeDtypeStruct((M, N), jnp.float32)
    return pl.pallas_call(
        body,
        out_shape=out_shape,
        in_specs=[
            pl.BlockSpec(memory_space=pltpu.VMEM),
            pl.BlockSpec(memory_space=pl.ANY),
            pl.BlockSpec(memory_space=pltpu.SMEM),
            pl.BlockSpec(memory_space=pltpu.SMEM),
        ],
        out_specs=pl.BlockSpec(memory_space=pl.ANY),
        scratch_shapes=[
            pltpu.VMEM((2, 2, M_CH, N_HALF), jnp.bfloat16),
            pltpu.VMEM((k_sh, N), jnp.bfloat16),
            pltpu.VMEM((k_sh, SUBW), jnp.float32),
            pltpu.VMEM((2, M_CH, N_HALF), jnp.bfloat16),
            pltpu.VMEM((2, M_CH, SUBW), jnp.float32),
            pltpu.SemaphoreType.DMA((2, N_SUB)),
            pltpu.SemaphoreType.DMA((2, N_SUB)),
            pltpu.SemaphoreType.DMA((2, N_SUB)),
            pltpu.SemaphoreType.DMA((2, N_SUB)),
            pltpu.SemaphoreType.DMA((2,)),
        ],
        compiler_params=pltpu.CompilerParams(
            collective_id=0, vmem_limit_bytes=60 * 1024 * 1024
        ),
    )(x, w_mat, scale_x, scale_w)
